# baseline (speedup 1.0000x reference)
import jax
import jax.numpy as jnp
from jax.experimental import pallas as pl
from jax.experimental.pallas import tpu as pltpu

M = 10000
K = 10000
N = 128
BM = 512


def _matmul_block(adj_ref, x_ref, out_ref):
    a = adj_ref[...].astype(jnp.bfloat16)
    b = x_ref[...].astype(jnp.bfloat16)
    out_ref[...] = jnp.dot(a, b, preferred_element_type=jnp.float32)


def kernel(x, adj):
    grid = (pl.cdiv(M, BM),)
    return pl.pallas_call(
        _matmul_block,
        grid=grid,
        in_specs=[
            pl.BlockSpec((BM, K), lambda i: (i, 0)),
            pl.BlockSpec((K, N), lambda i: (0, 0)),
        ],
        out_specs=pl.BlockSpec((BM, N), lambda i: (i, 0)),
        out_shape=jax.ShapeDtypeStruct((M, N), jnp.float32),
        compiler_params=pltpu.CompilerParams(
            dimension_semantics=("arbitrary",),
        ),
    )(adj, x)


# speedup vs baseline: 1.0105x; 1.0105x over previous
import jax
import jax.numpy as jnp
from jax.experimental import pallas as pl
from jax.experimental.pallas import tpu as pltpu

M = 10000
K = 10000
N = 128
BM = 400


def _matmul_block(adj_ref, x_ref, out_ref):
    a = adj_ref[...].astype(jnp.bfloat16)
    b = x_ref[...].astype(jnp.bfloat16)
    out_ref[...] = jnp.dot(a, b, preferred_element_type=jnp.float32)


def kernel(x, adj):
    grid = (pl.cdiv(M, BM),)
    return pl.pallas_call(
        _matmul_block,
        grid=grid,
        in_specs=[
            pl.BlockSpec((BM, K), lambda i: (i, 0)),
            pl.BlockSpec((K, N), lambda i: (0, 0)),
        ],
        out_specs=pl.BlockSpec((BM, N), lambda i: (i, 0)),
        out_shape=jax.ShapeDtypeStruct((M, N), jnp.float32),
        compiler_params=pltpu.CompilerParams(
            dimension_semantics=("arbitrary",),
        ),
    )(adj, x)
